# baseline (speedup 1.0000x reference)
import functools
from functools import partial

import jax
import jax.numpy as jnp
from jax import lax
from jax.experimental import pallas as pl
from jax.experimental.pallas import tpu as pltpu


def _pad_to(x, n, axis, value=0.0):
    pad = n - x.shape[axis]
    if pad <= 0:
        return x
    cfg = [(0, 0)] * x.ndim
    cfg[axis] = (0, pad)
    return jnp.pad(x, cfg, constant_values=value)


def _fps_body(npoint, n_real, x_ref, o_ref, dist_ref):
    R, L = dist_ref.shape
    x = x_ref[0, 0]
    y = x_ref[0, 1]
    z = x_ref[0, 2]
    fiota = (lax.broadcasted_iota(jnp.int32, (R, L), 0) * L
             + lax.broadcasted_iota(jnp.int32, (R, L), 1))
    real = fiota < n_real
    dist_ref[...] = jnp.where(real, 1e10, -1.0)
    o_ref[0, 0, 0] = 0

    def extract(sel, arr):
        return jnp.sum(jnp.where(sel, arr, 0.0))

    sel0 = fiota == 0
    lx0, ly0, lz0 = extract(sel0, x), extract(sel0, y), extract(sel0, z)

    def body(i, carry):
        lx, ly, lz = carry
        d = (x - lx) ** 2 + (y - ly) ** 2 + (z - lz) ** 2
        dist = jnp.minimum(dist_ref[...], d)
        dist_ref[...] = dist
        m = jnp.max(dist)
        eq = dist == m
        nxt = jnp.min(jnp.where(eq, fiota, jnp.int32(2 ** 30)))
        o_ref[0, 0, i] = nxt
        sel = fiota == nxt
        return (extract(sel, x), extract(sel, y), extract(sel, z))

    lax.fori_loop(1, npoint, body, (lx0, ly0, lz0))


def _fps(xyz, npoint):
    B, N, _ = xyz.shape
    NP = -(-N // 128) * 128
    R = NP // 128
    xt = jnp.swapaxes(_pad_to(xyz, NP, 1), 1, 2).reshape(B, 3, R, 128)
    return pl.pallas_call(
        partial(_fps_body, npoint, N),
        grid=(B,),
        in_specs=[pl.BlockSpec((1, 3, R, 128), lambda b: (b, 0, 0, 0))],
        out_specs=pl.BlockSpec((1, 1, npoint), lambda b: (b, 0, 0),
                               memory_space=pltpu.SMEM),
        out_shape=jax.ShapeDtypeStruct((B, 1, npoint), jnp.int32),
        scratch_shapes=[pltpu.VMEM((R, 128), jnp.float32)],
    )(xt).reshape(B, npoint)


def _bqmask_body(r2, c_ref, p_ref, o_ref):
    c8 = c_ref[0]
    pt = p_ref[0]
    SB = c8.shape[0]
    NC = pt.shape[1]
    cc = jnp.sum(c8 * c8, axis=1, keepdims=True)
    pp = jnp.sum(pt * pt, axis=0, keepdims=True)
    ab = jax.lax.dot(c8, pt, preferred_element_type=jnp.float32)
    d2 = jnp.maximum(cc + pp - 2.0 * ab, 0.0)
    maskf = jnp.where(d2 <= r2, 1.0, 0.0)
    piota = lax.broadcasted_iota(jnp.int32, (NC, NC // 16), 0)
    wiota = lax.broadcasted_iota(jnp.int32, (NC, NC // 16), 1)
    pm = jnp.where(piota // 16 == wiota,
                   (jnp.int32(1) << (piota % 16)).astype(jnp.float32), 0.0)
    o_ref[0] = jax.lax.dot(maskf, pm, preferred_element_type=jnp.float32)


def _bq_words(new_xyz, xyz, radius):
    B, S, _ = new_xyz.shape
    N = xyz.shape[1]
    NC = min(-(-N // 128) * 128, 2048)
    NP = -(-N // NC) * NC
    SB = min(S, 256)
    c8 = _pad_to(new_xyz, 8, 2)
    pt = jnp.swapaxes(_pad_to(_pad_to(xyz, NP, 1, 1e6), 8, 2), 1, 2)
    return pl.pallas_call(
        partial(_bqmask_body, radius * radius),
        grid=(B, S // SB, NP // NC),
        in_specs=[pl.BlockSpec((1, SB, 8), lambda b, s, c: (b, s, 0)),
                  pl.BlockSpec((1, 8, NC), lambda b, s, c: (b, 0, c))],
        out_specs=pl.BlockSpec((1, SB, NC // 16), lambda b, s, c: (b, s, c)),
        out_shape=jax.ShapeDtypeStruct((B, S, NP // 16), jnp.float32),
    )(c8, pt)


def _bq_compact_tmp(words, K, S, N, NT):
    B, _, NW = words.shape
    wi = words.astype(jnp.int32)
    bits = (wi[..., None] >> jnp.arange(16, dtype=jnp.int32)) & 1
    mask = (bits != 0).reshape(B, S, NW * 16)
    iota = jnp.arange(NW * 16, dtype=jnp.int32)
    rank = jnp.cumsum(mask.astype(jnp.int32), axis=-1) - mask.astype(jnp.int32)
    slot = jnp.where(mask & (rank < K), rank, K)
    out = jnp.full((B, S, K + 1), -1, jnp.int32)
    bi = jnp.arange(B)[:, None, None]
    si = jnp.arange(S)[None, :, None]
    out = out.at[bi, si, slot].set(
        jnp.broadcast_to(iota[None, None, :], mask.shape), mode='drop')
    out = out[..., :K]
    first = out[..., 0:1]
    out = jnp.where(out == -1, first, out)
    out = jnp.where(out == -1, 0, out)
    return out + (jnp.arange(B, dtype=jnp.int32) * NT)[:, None, None]


def _gather_rows_tmp(table, idx):
    return jnp.take(table, idx, axis=0)


def _mlp_body(radius, c_ref, g_ref, w1_ref, b1_ref, w2_ref, b2_ref,
              w3_ref, b3_ref, o_ref):
    SB, K, CT = g_ref.shape
    g = g_ref[...]
    cpad = c_ref[...]
    lane = lax.broadcasted_iota(jnp.int32, (SB, K, CT), 2)
    adj = jnp.where(lane < 3, (g - cpad[:, None, :]) / radius, g)
    h1 = jnp.maximum(jax.lax.dot(adj.reshape(SB * K, CT), w1_ref[...],
                                 preferred_element_type=jnp.float32)
                     + b1_ref[0][None, :], 0.0)
    h2 = jnp.maximum(jax.lax.dot(h1, w2_ref[...],
                                 preferred_element_type=jnp.float32)
                     + b2_ref[0][None, :], 0.0)
    h3 = jnp.maximum(jax.lax.dot(h2, w3_ref[...],
                                 preferred_element_type=jnp.float32)
                     + b3_ref[0][None, :], 0.0)
    C3 = h3.shape[-1]
    o_ref[...] = jnp.max(h3.reshape(SB, K, C3), axis=1)


def _mlp_pool(g, centers, weights, radius, K):
    BS, _, CT = g.shape
    (W1, b1), (W2, b2), (W3, b3) = weights
    C1, C2, C3 = W1.shape[1], W2.shape[1], W3.shape[1]
    w1cat = _pad_to(W1, CT, 0)
    cpad = _pad_to(centers, CT, 1)
    SB = min(BS, 128)
    return pl.pallas_call(
        partial(_mlp_body, radius),
        grid=(BS // SB,),
        in_specs=[pl.BlockSpec((SB, CT), lambda i: (i, 0)),
                  pl.BlockSpec((SB, K, CT), lambda i: (i, 0, 0)),
                  pl.BlockSpec((CT, C1), lambda i: (0, 0)),
                  pl.BlockSpec((1, C1), lambda i: (0, 0)),
                  pl.BlockSpec((C1, C2), lambda i: (0, 0)),
                  pl.BlockSpec((1, C2), lambda i: (0, 0)),
                  pl.BlockSpec((C2, C3), lambda i: (0, 0)),
                  pl.BlockSpec((1, C3), lambda i: (0, 0))],
        out_specs=pl.BlockSpec((SB, C3), lambda i: (i, 0)),
        out_shape=jax.ShapeDtypeStruct((BS, C3), jnp.float32),
    )(cpad, g, w1cat, b1.reshape(1, C1), W2, b2.reshape(1, C2),
      W3, b3.reshape(1, C3))


def _fp_body(u_ref, kt_ref, uf_ref, kf_ref, w1_ref, b1_ref, w2_ref, b2_ref,
             o_ref):
    u8 = u_ref[0]
    kt = kt_ref[0]
    S = u8.shape[0]
    M = kt.shape[1]
    uu = jnp.sum(u8 * u8, axis=1, keepdims=True)
    kk = jnp.sum(kt * kt, axis=0, keepdims=True)
    ab = jax.lax.dot(u8, kt, preferred_element_type=jnp.float32)
    d2 = jnp.maximum(uu + kk - 2.0 * ab, 0.0)
    miota = lax.broadcasted_iota(jnp.int32, (S, M), 1)
    kf = kf_ref[0]
    ws, neighs = [], []
    for _ in range(3):
        m = jnp.min(d2, axis=1, keepdims=True)
        sel = d2 == m
        idx = jnp.min(jnp.where(sel, miota, jnp.int32(2 ** 30)),
                      axis=1, keepdims=True)
        onehot = jnp.where(miota == idx, 1.0, 0.0)
        neighs.append(jax.lax.dot(onehot, kf,
                                  preferred_element_type=jnp.float32,
                                  precision=jax.lax.Precision.HIGHEST))
        ws.append(1.0 / (jnp.sqrt(jnp.maximum(m, 0.0)) + 1e-8))
        d2 = jnp.where(miota == idx, 3e38, d2)
    wsum = ws[0] + ws[1] + ws[2]
    interp = (neighs[0] * (ws[0] / wsum) + neighs[1] * (ws[1] / wsum)
              + neighs[2] * (ws[2] / wsum))
    h = jnp.concatenate([uf_ref[0], interp], axis=1)
    h = jnp.maximum(jax.lax.dot(h, w1_ref[...],
                                preferred_element_type=jnp.float32)
                    + b1_ref[0][None, :], 0.0)
    h = jnp.maximum(jax.lax.dot(h, w2_ref[...],
                                preferred_element_type=jnp.float32)
                    + b2_ref[0][None, :], 0.0)
    o_ref[0] = h


def _fp(unknown_xyz, known_xyz, unknown_feats, known_feats, weights):
    B, S, _ = unknown_xyz.shape
    M = known_xyz.shape[1]
    Cu = unknown_feats.shape[-1]
    Ck = known_feats.shape[-1]
    (W1, b1), (W2, b2) = weights
    CO = W2.shape[1]
    u8 = _pad_to(unknown_xyz, 8, 2)
    kt = jnp.swapaxes(_pad_to(known_xyz, 8, 2), 1, 2)
    return pl.pallas_call(
        _fp_body,
        grid=(B,),
        in_specs=[pl.BlockSpec((1, S, 8), lambda b: (b, 0, 0)),
                  pl.BlockSpec((1, 8, M), lambda b: (b, 0, 0)),
                  pl.BlockSpec((1, S, Cu), lambda b: (b, 0, 0)),
                  pl.BlockSpec((1, M, Ck), lambda b: (b, 0, 0)),
                  pl.BlockSpec(W1.shape, lambda b: (0, 0)),
                  pl.BlockSpec((1, W1.shape[1]), lambda b: (0, 0)),
                  pl.BlockSpec(W2.shape, lambda b: (0, 0)),
                  pl.BlockSpec((1, CO), lambda b: (0, 0))],
        out_specs=pl.BlockSpec((1, S, CO), lambda b: (b, 0, 0)),
        out_shape=jax.ShapeDtypeStruct((B, S, CO), jnp.float32),
    )(u8, kt, unknown_feats, known_feats, W1, b1.reshape(1, -1),
      W2, b2.reshape(1, -1))


def _sa_stage(xyz, feats, npoint, radius, K, weights):
    B, N, _ = xyz.shape
    fps_idx = _fps(xyz, npoint)
    if feats is None:
        CT = 16
        table = _pad_to(xyz, CT, 2)
    else:
        CT = -(-(3 + feats.shape[-1]) // 16) * 16
        table = _pad_to(jnp.concatenate([xyz, feats], axis=-1), CT, 2)
    table = table.reshape(B * N, CT)
    flat_fps = (fps_idx
                + (jnp.arange(B, dtype=jnp.int32) * N)[:, None]).reshape(-1)
    new_xyz = _gather_rows_tmp(table, flat_fps).reshape(B, npoint, CT)[..., :3]
    words = _bq_words(new_xyz, xyz, radius)
    bq = _bq_compact_tmp(words, K, npoint, N, N)
    g = _gather_rows_tmp(table, bq.reshape(-1))
    f = _mlp_pool(g.reshape(B * npoint, K, CT),
                  new_xyz.reshape(B * npoint, 3), weights, radius, K)
    return new_xyz, f.reshape(B, npoint, -1), fps_idx


def kernel(pointcloud, params):
    xyz = pointcloud[:, :, 0:3]
    sa1_xyz, sa1_f, sa1_inds = _sa_stage(xyz, None, 2048, 0.2, 64,
                                         params['sa1'])
    sa2_xyz, sa2_f, sa2_inds = _sa_stage(sa1_xyz, sa1_f, 1024, 0.4, 32,
                                         params['sa2'])
    sa3_xyz, sa3_f, _ = _sa_stage(sa2_xyz, sa2_f, 512, 0.8, 16, params['sa3'])
    sa4_xyz, sa4_f, _ = _sa_stage(sa3_xyz, sa3_f, 256, 1.2, 16, params['sa4'])
    fp1_f = _fp(sa3_xyz, sa4_xyz, sa3_f, sa4_f, params['fp1'])
    fp2_f = _fp(sa2_xyz, sa3_xyz, sa2_f, fp1_f, params['fp2'])
    fp2_inds = sa1_inds[:, :sa2_inds.shape[1]]
    return fp2_f, sa2_xyz, fp2_inds, sa4_xyz, sa4_f


# speedup vs baseline: 60.7817x; 60.7817x over previous
import functools
from functools import partial

import jax
import jax.numpy as jnp
from jax import lax
from jax.experimental import pallas as pl
from jax.experimental.pallas import tpu as pltpu


def _pad_to(x, n, axis, value=0.0):
    pad = n - x.shape[axis]
    if pad <= 0:
        return x
    cfg = [(0, 0)] * x.ndim
    cfg[axis] = (0, pad)
    return jnp.pad(x, cfg, constant_values=value)


def _fps_body(npoint, n_real, x_ref, o_ref, dist_ref):
    R, L = dist_ref.shape
    x = x_ref[0, 0]
    y = x_ref[0, 1]
    z = x_ref[0, 2]
    fiota = (lax.broadcasted_iota(jnp.int32, (R, L), 0) * L
             + lax.broadcasted_iota(jnp.int32, (R, L), 1))
    real = fiota < n_real
    dist_ref[...] = jnp.where(real, 1e10, -1.0)
    o_ref[0, 0, 0] = 0

    def extract(sel, arr):
        return jnp.sum(jnp.where(sel, arr, 0.0))

    sel0 = fiota == 0
    lx0, ly0, lz0 = extract(sel0, x), extract(sel0, y), extract(sel0, z)

    def body(i, carry):
        lx, ly, lz = carry
        d = (x - lx) ** 2 + (y - ly) ** 2 + (z - lz) ** 2
        dist = jnp.minimum(dist_ref[...], d)
        dist_ref[...] = dist
        m = jnp.max(dist)
        eq = dist == m
        nxt = jnp.min(jnp.where(eq, fiota, jnp.int32(2 ** 30)))
        o_ref[0, 0, i] = nxt
        sel = fiota == nxt
        return (extract(sel, x), extract(sel, y), extract(sel, z))

    lax.fori_loop(1, npoint, body, (lx0, ly0, lz0))


def _fps(xyz, npoint):
    B, N, _ = xyz.shape
    NP = -(-N // 128) * 128
    R = NP // 128
    xt = jnp.swapaxes(_pad_to(xyz, NP, 1), 1, 2).reshape(B, 3, R, 128)
    return pl.pallas_call(
        partial(_fps_body, npoint, N),
        grid=(B,),
        in_specs=[pl.BlockSpec((1, 3, R, 128), lambda b: (b, 0, 0, 0))],
        out_specs=pl.BlockSpec((1, 1, npoint), lambda b: (b, 0, 0),
                               memory_space=pltpu.SMEM),
        out_shape=jax.ShapeDtypeStruct((B, 1, npoint), jnp.int32),
        scratch_shapes=[pltpu.VMEM((R, 128), jnp.float32)],
    )(xt).reshape(B, npoint)


def _bqmask_body(r2, c_ref, p_ref, o_ref):
    c8 = c_ref[0]
    pt = p_ref[0]
    SB = c8.shape[0]
    NC = pt.shape[1]
    cc = jnp.sum(c8 * c8, axis=1, keepdims=True)
    pp = jnp.sum(pt * pt, axis=0, keepdims=True)
    ab = jax.lax.dot(c8, pt, preferred_element_type=jnp.float32)
    d2 = jnp.maximum(cc + pp - 2.0 * ab, 0.0)
    maskf = jnp.where(d2 <= r2, 1.0, 0.0)
    piota = lax.broadcasted_iota(jnp.int32, (NC, NC // 16), 0)
    wiota = lax.broadcasted_iota(jnp.int32, (NC, NC // 16), 1)
    pm = jnp.where(piota // 16 == wiota,
                   (jnp.int32(1) << (piota % 16)).astype(jnp.float32), 0.0)
    o_ref[0] = jax.lax.dot(maskf, pm, preferred_element_type=jnp.float32)


def _bq_words(new_xyz, xyz, radius):
    B, S, _ = new_xyz.shape
    N = xyz.shape[1]
    NC = min(-(-N // 128) * 128, 2048)
    NP = -(-N // NC) * NC
    SB = min(S, 256)
    c8 = _pad_to(new_xyz, 8, 2)
    pt = jnp.swapaxes(_pad_to(_pad_to(xyz, NP, 1, 1e6), 8, 2), 1, 2)
    return pl.pallas_call(
        partial(_bqmask_body, radius * radius),
        grid=(B, S // SB, NP // NC),
        in_specs=[pl.BlockSpec((1, SB, 8), lambda b, s, c: (b, s, 0)),
                  pl.BlockSpec((1, 8, NC), lambda b, s, c: (b, 0, c))],
        out_specs=pl.BlockSpec((1, SB, NC // 16), lambda b, s, c: (b, s, c)),
        out_shape=jax.ShapeDtypeStruct((B, S, NP // 16), jnp.float32),
    )(c8, pt)


def _compact_body(K, NT, S, c_ref, o_ref):
    _, SB, NW = c_ref.shape
    wf = c_ref[0]
    wi = wf.astype(jnp.int32)
    v = wi - ((wi >> 1) & 0x5555)
    v = (v & 0x3333) + ((v >> 2) & 0x3333)
    v = (v + (v >> 4)) & 0x0F0F
    v = (v + (v >> 8)) & 0x1F
    cnt = v.astype(jnp.float32)
    wio = lax.broadcasted_iota(jnp.int32, (NW, NW), 0)
    wio2 = lax.broadcasted_iota(jnp.int32, (NW, NW), 1)
    lt = jnp.where(wio < wio2, 1.0, 0.0)
    excl = jax.lax.dot(cnt, lt, preferred_element_type=jnp.float32,
                       precision=jax.lax.Precision.HIGHEST)
    total = jnp.sum(cnt, axis=1, keepdims=True)
    lane_f = lax.broadcasted_iota(jnp.int32, (SB, NW), 1).astype(jnp.float32)
    boff = pl.program_id(0) * NT
    cols = []
    first = None
    for j in range(K):
        jf = float(j)
        sel = jnp.where((excl <= jf) & (excl + cnt > jf), 1.0, 0.0)
        wv = jnp.sum(sel * wf, axis=1, keepdims=True)
        wbase = jnp.sum(sel * lane_f, axis=1, keepdims=True)
        er = jnp.sum(sel * excl, axis=1, keepdims=True)
        wvi = wv.astype(jnp.int32)
        r = j - er.astype(jnp.int32)
        c = jnp.zeros_like(r)
        pos = jnp.zeros_like(r)
        for bbit in range(16):
            c = c + ((wvi >> bbit) & 1)
            pos = pos + jnp.where(c <= r, 1, 0)
        col = wbase.astype(jnp.int32) * 16 + pos + boff
        if first is None:
            first = jnp.where(total > 0.0, col, boff)
        col = jnp.where(total > jf, col, first)
        cols.append(col)
    o_ref[0] = jnp.concatenate(cols, axis=1)


def _bq_compact(words, K, S, NT):
    B, S_, NW = words.shape
    SB = min(S_, 256)
    return pl.pallas_call(
        partial(_compact_body, K, NT, S_),
        grid=(B, S_ // SB),
        in_specs=[pl.BlockSpec((1, SB, NW), lambda b, s: (b, s, 0))],
        out_specs=pl.BlockSpec((1, SB, K), lambda b, s: (b, s, 0)),
        out_shape=jax.ShapeDtypeStruct((B, S_, K), jnp.int32),
    )(words)


def _sc_gather(table, idx):
    R, = idx.shape
    D = table.shape[1]
    from jax.experimental.pallas import tpu_sc as plsc
    info = plsc.get_sparse_core_info()
    NWK = info.num_cores * info.num_subcores
    b_per_w = R // NWK
    chunk = min(b_per_w, 128)
    n_chunks = b_per_w // chunk
    mesh = plsc.VectorSubcoreMesh(core_axis_name="c", subcore_axis_name="s")

    @functools.partial(
        pl.kernel,
        out_type=jax.ShapeDtypeStruct((R, D), jnp.float32),
        mesh=mesh,
        compiler_params=pltpu.CompilerParams(use_tc_tiling_on_sc=False),
        scratch_types=[pltpu.VMEM((chunk,), jnp.int32),
                       pltpu.VMEM((chunk, D), jnp.float32),
                       pltpu.SemaphoreType.DMA],
    )
    def k(table_hbm, idx_hbm, out_hbm, idx_v, rows_v, sem):
        wid = lax.axis_index("s") * info.num_cores + lax.axis_index("c")

        def body(ci, _):
            base = wid * b_per_w + ci * chunk
            pltpu.sync_copy(idx_hbm.at[pl.ds(base, chunk)], idx_v)
            pltpu.async_copy(table_hbm.at[idx_v], rows_v, sem).wait()
            pltpu.sync_copy(rows_v, out_hbm.at[pl.ds(base, chunk)])
            return 0

        lax.fori_loop(0, n_chunks, body, 0)

    return k(table, idx)


def _mlp_body(radius, c_ref, g_ref, w1_ref, b1_ref, w2_ref, b2_ref,
              w3_ref, b3_ref, o_ref):
    SB, K, CT = g_ref.shape
    g = g_ref[...]
    cpad = c_ref[...]
    lane = lax.broadcasted_iota(jnp.int32, (SB, K, CT), 2)
    adj = jnp.where(lane < 3, (g - cpad[:, None, :]) / radius, g)
    h1 = jnp.maximum(jax.lax.dot(adj.reshape(SB * K, CT), w1_ref[...],
                                 preferred_element_type=jnp.float32)
                     + b1_ref[0][None, :], 0.0)
    h2 = jnp.maximum(jax.lax.dot(h1, w2_ref[...],
                                 preferred_element_type=jnp.float32)
                     + b2_ref[0][None, :], 0.0)
    h3 = jnp.maximum(jax.lax.dot(h2, w3_ref[...],
                                 preferred_element_type=jnp.float32)
                     + b3_ref[0][None, :], 0.0)
    C3 = h3.shape[-1]
    o_ref[...] = jnp.max(h3.reshape(SB, K, C3), axis=1)


def _mlp_pool(g, centers, weights, radius, K):
    BS, _, CT = g.shape
    (W1, b1), (W2, b2), (W3, b3) = weights
    C1, C2, C3 = W1.shape[1], W2.shape[1], W3.shape[1]
    w1cat = _pad_to(W1, CT, 0)
    cpad = _pad_to(centers, CT, 1)
    SB = min(BS, 128)
    return pl.pallas_call(
        partial(_mlp_body, radius),
        grid=(BS // SB,),
        in_specs=[pl.BlockSpec((SB, CT), lambda i: (i, 0)),
                  pl.BlockSpec((SB, K, CT), lambda i: (i, 0, 0)),
                  pl.BlockSpec((CT, C1), lambda i: (0, 0)),
                  pl.BlockSpec((1, C1), lambda i: (0, 0)),
                  pl.BlockSpec((C1, C2), lambda i: (0, 0)),
                  pl.BlockSpec((1, C2), lambda i: (0, 0)),
                  pl.BlockSpec((C2, C3), lambda i: (0, 0)),
                  pl.BlockSpec((1, C3), lambda i: (0, 0))],
        out_specs=pl.BlockSpec((SB, C3), lambda i: (i, 0)),
        out_shape=jax.ShapeDtypeStruct((BS, C3), jnp.float32),
    )(cpad, g, w1cat, b1.reshape(1, C1), W2, b2.reshape(1, C2),
      W3, b3.reshape(1, C3))


def _fp_body(u_ref, kt_ref, uf_ref, kf_ref, w1_ref, b1_ref, w2_ref, b2_ref,
             o_ref):
    u8 = u_ref[0]
    kt = kt_ref[0]
    S = u8.shape[0]
    M = kt.shape[1]
    uu = jnp.sum(u8 * u8, axis=1, keepdims=True)
    kk = jnp.sum(kt * kt, axis=0, keepdims=True)
    ab = jax.lax.dot(u8, kt, preferred_element_type=jnp.float32)
    d2 = jnp.maximum(uu + kk - 2.0 * ab, 0.0)
    miota = lax.broadcasted_iota(jnp.int32, (S, M), 1)
    kf = kf_ref[0]
    ws, neighs = [], []
    for _ in range(3):
        m = jnp.min(d2, axis=1, keepdims=True)
        sel = d2 == m
        idx = jnp.min(jnp.where(sel, miota, jnp.int32(2 ** 30)),
                      axis=1, keepdims=True)
        onehot = jnp.where(miota == idx, 1.0, 0.0)
        neighs.append(jax.lax.dot(onehot, kf,
                                  preferred_element_type=jnp.float32,
                                  precision=jax.lax.Precision.HIGHEST))
        ws.append(1.0 / (jnp.sqrt(jnp.maximum(m, 0.0)) + 1e-8))
        d2 = jnp.where(miota == idx, 3e38, d2)
    wsum = ws[0] + ws[1] + ws[2]
    interp = (neighs[0] * (ws[0] / wsum) + neighs[1] * (ws[1] / wsum)
              + neighs[2] * (ws[2] / wsum))
    h = jnp.concatenate([uf_ref[0], interp], axis=1)
    h = jnp.maximum(jax.lax.dot(h, w1_ref[...],
                                preferred_element_type=jnp.float32)
                    + b1_ref[0][None, :], 0.0)
    h = jnp.maximum(jax.lax.dot(h, w2_ref[...],
                                preferred_element_type=jnp.float32)
                    + b2_ref[0][None, :], 0.0)
    o_ref[0] = h


def _fp(unknown_xyz, known_xyz, unknown_feats, known_feats, weights):
    B, S, _ = unknown_xyz.shape
    M = known_xyz.shape[1]
    Cu = unknown_feats.shape[-1]
    Ck = known_feats.shape[-1]
    (W1, b1), (W2, b2) = weights
    CO = W2.shape[1]
    u8 = _pad_to(unknown_xyz, 8, 2)
    kt = jnp.swapaxes(_pad_to(known_xyz, 8, 2), 1, 2)
    return pl.pallas_call(
        _fp_body,
        grid=(B,),
        in_specs=[pl.BlockSpec((1, S, 8), lambda b: (b, 0, 0)),
                  pl.BlockSpec((1, 8, M), lambda b: (b, 0, 0)),
                  pl.BlockSpec((1, S, Cu), lambda b: (b, 0, 0)),
                  pl.BlockSpec((1, M, Ck), lambda b: (b, 0, 0)),
                  pl.BlockSpec(W1.shape, lambda b: (0, 0)),
                  pl.BlockSpec((1, W1.shape[1]), lambda b: (0, 0)),
                  pl.BlockSpec(W2.shape, lambda b: (0, 0)),
                  pl.BlockSpec((1, CO), lambda b: (0, 0))],
        out_specs=pl.BlockSpec((1, S, CO), lambda b: (b, 0, 0)),
        out_shape=jax.ShapeDtypeStruct((B, S, CO), jnp.float32),
    )(u8, kt, unknown_feats, known_feats, W1, b1.reshape(1, -1),
      W2, b2.reshape(1, -1))


def _sa_stage(xyz, feats, npoint, radius, K, weights):
    B, N, _ = xyz.shape
    fps_idx = _fps(xyz, npoint)
    if feats is None:
        CT = 16
        table = _pad_to(xyz, CT, 2)
    else:
        CT = -(-(3 + feats.shape[-1]) // 16) * 16
        table = _pad_to(jnp.concatenate([xyz, feats], axis=-1), CT, 2)
    table = table.reshape(B * N, CT)
    flat_fps = (fps_idx
                + (jnp.arange(B, dtype=jnp.int32) * N)[:, None]).reshape(-1)
    new_xyz = _sc_gather(table, flat_fps).reshape(B, npoint, CT)[..., :3]
    words = _bq_words(new_xyz, xyz, radius)
    bq = _bq_compact(words, K, npoint, N)
    g = _sc_gather(table, bq.reshape(-1))
    f = _mlp_pool(g.reshape(B * npoint, K, CT),
                  new_xyz.reshape(B * npoint, 3), weights, radius, K)
    return new_xyz, f.reshape(B, npoint, -1), fps_idx


def kernel(pointcloud, params):
    xyz = pointcloud[:, :, 0:3]
    sa1_xyz, sa1_f, sa1_inds = _sa_stage(xyz, None, 2048, 0.2, 64,
                                         params['sa1'])
    sa2_xyz, sa2_f, sa2_inds = _sa_stage(sa1_xyz, sa1_f, 1024, 0.4, 32,
                                         params['sa2'])
    sa3_xyz, sa3_f, _ = _sa_stage(sa2_xyz, sa2_f, 512, 0.8, 16, params['sa3'])
    sa4_xyz, sa4_f, _ = _sa_stage(sa3_xyz, sa3_f, 256, 1.2, 16, params['sa4'])
    fp1_f = _fp(sa3_xyz, sa4_xyz, sa3_f, sa4_f, params['fp1'])
    fp2_f = _fp(sa2_xyz, sa3_xyz, sa2_f, fp1_f, params['fp2'])
    fp2_inds = sa1_inds[:, :sa2_inds.shape[1]]
    return fp2_f, sa2_xyz, fp2_inds, sa4_xyz, sa4_f
